# baseline (speedup 1.0000x reference)
import functools

import jax
import jax.numpy as jnp
from jax import lax
from jax.experimental import pallas as pl
from jax.experimental.pallas import tpu as pltpu
from jax.experimental.pallas import tpu_sc as plsc

NUM_EMB = 1_000_000
VEC = 128
RED = 32
BATCH = 4096
SEQ = 200


PACK = 4
ROWS_P = NUM_EMB // PACK
QBLK = 2000
NBLK = ROWS_P // QBLK


def _proj_body(t0, t1, t2, t3, w_ref, b_ref, o_ref):
    parts = []
    for t in (t0, t1, t2, t3):
        parts.append(
            lax.dot_general(
                t[...], w_ref[...],
                dimension_numbers=(((1,), (1,)), ((), ())),
                preferred_element_type=jnp.float32,
            )
        )
    o_ref[...] = jnp.concatenate(parts, axis=1) + b_ref[...]


def _project(table, W, b):
    b4 = jnp.tile(b, PACK).reshape(1, PACK * RED)
    t_spec = lambda q: pl.BlockSpec((QBLK, VEC), lambda i, q=q: (q * NBLK + i, 0))
    return pl.pallas_call(
        _proj_body,
        grid=(NBLK,),
        in_specs=[
            t_spec(0), t_spec(1), t_spec(2), t_spec(3),
            pl.BlockSpec((RED, VEC), lambda i: (0, 0)),
            pl.BlockSpec((1, PACK * RED), lambda i: (0, 0)),
        ],
        out_specs=pl.BlockSpec((QBLK, PACK * RED), lambda i: (i, 0)),
        out_shape=jax.ShapeDtypeStruct((ROWS_P, PACK * RED), jnp.float32),
    )(table, table, table, table, W, b4)


_B = BATCH * SEQ
_NW = 32
_BPW = _B // _NW
_G = 128
_KG = 8
_C = _G * _KG
_NCHUNK = _BPW // _C
_L = 16


_NST = SEQ // _KG


def _remap_slab(slab_v):
    def row(st, carry):
        for k in range(_KG * _G // _L):
            g = slab_v[st, pl.ds(k * _L, _L)]
            q = (
                jnp.where(g >= ROWS_P, 1, 0)
                + jnp.where(g >= 2 * ROWS_P, 1, 0)
                + jnp.where(g >= 3 * ROWS_P, 1, 0)
            ).astype(jnp.int32)
            slab_v[st, pl.ds(k * _L, _L)] = (g << 2) - q * (NUM_EMB - 1)
        return carry

    lax.fori_loop(0, _NST, row, 0)


_GW = 64
_CH = _KG * _GW
_NBT = BATCH // VEC


def _make_gather_body(h):
    def body(xq_hbm, proj_hbm, out_hbm, slab_v, dst_v, rows_v, gsem, ssem):
        cid = lax.axis_index("c")
        sid = lax.axis_index("s")
        wid = sid * 2 + cid
        btl = wid // 2
        e = wid - btl * 2
        base = btl * (SEQ * VEC)

        col = pl.multiple_of((h * (_NBT // 2) + btl) * _C, _C)
        pltpu.sync_copy(xq_hbm.at[pl.ds(0, _NST), pl.ds(col, _C)], slab_v)
        _remap_slab(slab_v)

        iota = lax.iota(jnp.int32, _L)

        def fire_g(ci, buf):
            return [
                pltpu.async_copy(
                    proj_hbm.at[slab_v.at[ci, pl.ds(j * _G + e * _GW, _GW)]],
                    rows_v.at[buf, pl.ds(j * _GW, _GW)],
                    gsem,
                )
                for j in range(_KG)
            ]

        def fire_s(ci, buf):
            for j in range(_KG):
                for k in range(_GW // _L):
                    bl = e * _GW + k * _L + iota
                    s = ci * _KG + j
                    si = s >> 2
                    dst_v[buf, j, pl.ds(k * _L, _L)] = (
                        base + (si << 9) + (bl << 2) + (s - (si << 2))
                    )
            return [
                pltpu.async_copy(
                    rows_v.at[buf, pl.ds(j * _GW, _GW)],
                    out_hbm.at[dst_v.at[buf, j]],
                    ssem,
                )
                for j in range(_KG)
            ]

        def drain(copies):
            for cp in copies:
                cp.wait()

        drain(fire_g(0, 0))

        def pair(st2, carry):
            c0 = st2 * 2
            ss0 = fire_s(c0, 0)
            gs1 = fire_g(c0 + 1, 1)
            drain(gs1)
            drain(ss0)
            ss1 = fire_s(c0 + 1, 1)
            gs2 = fire_g(c0 + 2, 0)
            drain(gs2)
            drain(ss1)
            return carry

        lax.fori_loop(0, _NST // 2, pair, 0)
        drain(fire_s(_NST - 1, 0))

    return body


@functools.cache
def _gather_kernel(h):
    return pl.kernel(
        _make_gather_body(h),
        mesh=plsc.VectorSubcoreMesh(core_axis_name="c", subcore_axis_name="s"),
        compiler_params=pltpu.CompilerParams(use_tc_tiling_on_sc=False),
        out_type=jax.ShapeDtypeStruct((_B // 2, RED), jnp.float32),
        scratch_types=[
            pltpu.VMEM((_NST, _C), jnp.int32),
            pltpu.VMEM((2, _KG, _GW), jnp.int32),
            pltpu.VMEM((2, _CH, RED), jnp.float32),
            pltpu.SemaphoreType.DMA,
            pltpu.SemaphoreType.DMA,
        ],
    )


_BT = BATCH // VEC
_SR = SEQ * RED
_PB = _B // PACK


def _tr_body(t_ref, o_ref):
    x3 = t_ref[0].reshape(_SR // VEC, VEC, VEC)
    o_ref[...] = x3.transpose(0, 2, 1).reshape(_SR, VEC)


def _tr2_body(t_ref, prev_ref, o_ref):
    del prev_ref
    x3 = t_ref[0].reshape(_SR // VEC, VEC, VEC)
    o_ref[...] = x3.transpose(0, 2, 1).reshape(_SR, VEC)


def _transpose_half(flat_h, prev=None):
    x = flat_h.reshape(_NBT // 2, _SR, VEC)
    if prev is None:
        return pl.pallas_call(
            _tr_body,
            grid=(_NBT // 2,),
            in_specs=[pl.BlockSpec((1, _SR, VEC), lambda i: (i, 0, 0))],
            out_specs=pl.BlockSpec((_SR, VEC), lambda i: (0, i)),
            out_shape=jax.ShapeDtypeStruct((_SR, BATCH), jnp.float32),
        )(x)
    return pl.pallas_call(
        _tr2_body,
        grid=(_NBT // 2,),
        in_specs=[
            pl.BlockSpec((1, _SR, VEC), lambda i: (i, 0, 0)),
            pl.BlockSpec(memory_space=pl.ANY),
        ],
        out_specs=pl.BlockSpec((_SR, VEC), lambda i: (0, i + _NBT // 2)),
        out_shape=jax.ShapeDtypeStruct((_SR, BATCH), jnp.float32),
        input_output_aliases={1: 0},
    )(x, prev)


def kernel(x, table, W, b):
    proj = _project(table, W, b).reshape(NUM_EMB, RED)
    xq = (
        x.astype(jnp.int32)
        .transpose(1, 0)
        .reshape(_NST, _KG, BATCH // VEC, VEC)
        .transpose(0, 2, 1, 3)
        .reshape(_NST, BATCH // VEC * _KG * VEC)
    )
    g0 = _gather_kernel(0)(xq, proj)
    g1 = _gather_kernel(1)(xq, proj)
    t1 = _transpose_half(g0.reshape(_PB // 2, VEC))
    out2 = _transpose_half(g1.reshape(_PB // 2, VEC), prev=t1)
    return out2.reshape(SEQ, RED, BATCH).transpose(2, 0, 1)


# speedup vs baseline: 1.1242x; 1.1242x over previous
import functools

import jax
import jax.numpy as jnp
from jax import lax
from jax.experimental import pallas as pl
from jax.experimental.pallas import tpu as pltpu
from jax.experimental.pallas import tpu_sc as plsc

NUM_EMB = 1_000_000
VEC = 128
RED = 32
BATCH = 4096
SEQ = 200


PACK = 4
ROWS_P = NUM_EMB // PACK
QBLK = 5000
NBLK = ROWS_P // QBLK


def _proj_body(t0, t1, t2, t3, w_ref, b_ref, o_ref):
    parts = []
    for t in (t0, t1, t2, t3):
        parts.append(
            lax.dot_general(
                t[...], w_ref[...],
                dimension_numbers=(((1,), (1,)), ((), ())),
                preferred_element_type=jnp.float32,
            )
        )
    o_ref[...] = jnp.concatenate(parts, axis=1) + b_ref[...]


def _project(table, W, b):
    b4 = jnp.tile(b, PACK).reshape(1, PACK * RED)
    t_spec = lambda q: pl.BlockSpec((QBLK, VEC), lambda i, q=q: (q * NBLK + i, 0))
    return pl.pallas_call(
        _proj_body,
        grid=(NBLK,),
        in_specs=[
            t_spec(0), t_spec(1), t_spec(2), t_spec(3),
            pl.BlockSpec((RED, VEC), lambda i: (0, 0)),
            pl.BlockSpec((1, PACK * RED), lambda i: (0, 0)),
        ],
        out_specs=pl.BlockSpec((QBLK, PACK * RED), lambda i: (i, 0)),
        out_shape=jax.ShapeDtypeStruct((ROWS_P, PACK * RED), jnp.float32),
    )(table, table, table, table, W, b4)


_B = BATCH * SEQ
_NW = 32
_BPW = _B // _NW
_G = 128
_KG = 8
_C = _G * _KG
_NCHUNK = _BPW // _C
_L = 16


_NST = SEQ // _KG


def _remap_slab(slab_v):
    def row(st, carry):
        for k in range(_KG * _G // _L):
            g = slab_v[st, pl.ds(k * _L, _L)]
            q = (
                jnp.where(g >= ROWS_P, 1, 0)
                + jnp.where(g >= 2 * ROWS_P, 1, 0)
                + jnp.where(g >= 3 * ROWS_P, 1, 0)
            ).astype(jnp.int32)
            slab_v[st, pl.ds(k * _L, _L)] = (g << 2) - q * (NUM_EMB - 1)
        return carry

    lax.fori_loop(0, _NST, row, 0)


_NBT = BATCH // VEC


def _gather_body(xq_hbm, proj_hbm, out_hbm, slab_v, dst_v, rows_v, gsem, ssem):
    cid = lax.axis_index("c")
    sid = lax.axis_index("s")
    wid = sid * 2 + cid
    base = wid * _BPW

    col = pl.multiple_of(wid * _C, _C)
    pltpu.sync_copy(xq_hbm.at[pl.ds(0, _NST), pl.ds(col, _C)], slab_v)
    _remap_slab(slab_v)

    iota = lax.iota(jnp.int32, _L)

    def fire_g(ci, buf):
        return [
            pltpu.async_copy(
                proj_hbm.at[slab_v.at[ci, pl.ds(j * _G, _G)]],
                rows_v.at[buf, pl.ds(j * _G, _G)],
                gsem,
            )
            for j in range(_KG)
        ]

    def fire_s(ci, buf):
        for j in range(_KG):
            for k in range(_G // _L):
                bl = k * _L + iota
                s = ci * _KG + j
                si = s >> 2
                dst_v[buf, j, pl.ds(k * _L, _L)] = (
                    base + (si << 9) + (bl << 2) + (s - (si << 2))
                )
        return [
            pltpu.async_copy(
                rows_v.at[buf, pl.ds(j * _G, _G)],
                out_hbm.at[dst_v.at[buf, j]],
                ssem,
            )
            for j in range(_KG)
        ]

    def drain(copies):
        for cp in copies:
            cp.wait()

    drain(fire_g(0, 0))

    def pair(st2, carry):
        c0 = st2 * 2
        ss0 = fire_s(c0, 0)
        gs1 = fire_g(c0 + 1, 1)
        drain(gs1)
        drain(ss0)
        ss1 = fire_s(c0 + 1, 1)
        gs2 = fire_g(c0 + 2, 0)
        drain(gs2)
        drain(ss1)
        return carry

    lax.fori_loop(0, _NST // 2, pair, 0)
    drain(fire_s(_NST - 1, 0))


@functools.cache
def _gather_kernel():
    return pl.kernel(
        _gather_body,
        mesh=plsc.VectorSubcoreMesh(core_axis_name="c", subcore_axis_name="s"),
        compiler_params=pltpu.CompilerParams(use_tc_tiling_on_sc=False),
        out_type=jax.ShapeDtypeStruct((_B, RED), jnp.float32),
        scratch_types=[
            pltpu.VMEM((_NST, _C), jnp.int32),
            pltpu.VMEM((2, _KG, _G), jnp.int32),
            pltpu.VMEM((2, _C, RED), jnp.float32),
            pltpu.SemaphoreType.DMA,
            pltpu.SemaphoreType.DMA,
        ],
    )


_BT = BATCH // VEC
_SR = SEQ * RED
_PB = _B // PACK


def _tr_body(t_ref, o_ref):
    x3 = t_ref[0].reshape(_SR // VEC, VEC, VEC)
    o_ref[...] = x3.transpose(0, 2, 1).reshape(_SR, VEC)


def _transpose(out_flat):
    x = out_flat.reshape(_NBT, _SR, VEC)
    return pl.pallas_call(
        _tr_body,
        grid=(_NBT,),
        in_specs=[pl.BlockSpec((1, _SR, VEC), lambda i: (i, 0, 0))],
        out_specs=pl.BlockSpec((_SR, VEC), lambda i: (0, i)),
        out_shape=jax.ShapeDtypeStruct((_SR, BATCH), jnp.float32),
    )(x)


def kernel(x, table, W, b):
    proj = _project(table, W, b).reshape(NUM_EMB, RED)
    xq = (
        x.astype(jnp.int32)
        .transpose(1, 0)
        .reshape(_NST, _KG, BATCH // VEC, VEC)
        .transpose(0, 2, 1, 3)
        .reshape(_NST, BATCH // VEC * _KG * VEC)
    )
    out_flat = _gather_kernel()(xq, proj)
    out2 = _transpose(out_flat.reshape(_PB, VEC))
    return out2.reshape(SEQ, RED, BATCH).transpose(2, 0, 1)


# speedup vs baseline: 1.1370x; 1.0114x over previous
import functools

import jax
import jax.numpy as jnp
from jax import lax
from jax.experimental import pallas as pl
from jax.experimental.pallas import tpu as pltpu
from jax.experimental.pallas import tpu_sc as plsc

NUM_EMB = 1_000_000
VEC = 128
RED = 32
BATCH = 4096
SEQ = 200


PACK = 4
ROWS_P = NUM_EMB // PACK
QBLK = 5000
NBLK = ROWS_P // QBLK


def _proj_body(t0, t1, t2, t3, w_ref, b_ref, o_ref):
    parts = []
    for t in (t0, t1, t2, t3):
        parts.append(
            lax.dot_general(
                t[...], w_ref[...],
                dimension_numbers=(((1,), (1,)), ((), ())),
                preferred_element_type=jnp.float32,
            )
        )
    o_ref[...] = jnp.concatenate(parts, axis=1) + b_ref[...]


def _project(table, W, b):
    b4 = jnp.tile(b, PACK).reshape(1, PACK * RED)
    t_spec = lambda q: pl.BlockSpec((QBLK, VEC), lambda i, q=q: (q * NBLK + i, 0))
    return pl.pallas_call(
        _proj_body,
        grid=(NBLK,),
        in_specs=[
            t_spec(0), t_spec(1), t_spec(2), t_spec(3),
            pl.BlockSpec((RED, VEC), lambda i: (0, 0)),
            pl.BlockSpec((1, PACK * RED), lambda i: (0, 0)),
        ],
        out_specs=pl.BlockSpec((QBLK, PACK * RED), lambda i: (i, 0)),
        out_shape=jax.ShapeDtypeStruct((ROWS_P, PACK * RED), jnp.float32),
    )(table, table, table, table, W, b4)


_B = BATCH * SEQ
_NW = 32
_BPW = _B // _NW
_G = 128
_KG = 8
_C = _G * _KG
_NCHUNK = _BPW // _C
_L = 16


_NST = SEQ // _KG


def _remap_slab(slab_v):
    def row(st, carry):
        for k in range(_KG * _G // _L):
            g = slab_v[st, pl.ds(k * _L, _L)]
            q = (
                jnp.where(g >= ROWS_P, 1, 0)
                + jnp.where(g >= 2 * ROWS_P, 1, 0)
                + jnp.where(g >= 3 * ROWS_P, 1, 0)
            ).astype(jnp.int32)
            slab_v[st, pl.ds(k * _L, _L)] = (g << 2) - q * (NUM_EMB - 1)
        return carry

    lax.fori_loop(0, _NST, row, 0)


_NBT = BATCH // VEC


def _gather_body(xq_hbm, proj_hbm, out_hbm, slab_v, dst_v, rows_v, gsem, ssem):
    cid = lax.axis_index("c")
    sid = lax.axis_index("s")
    wid = sid * 2 + cid
    base = wid * _BPW

    col = pl.multiple_of(wid * _C, _C)
    pltpu.sync_copy(xq_hbm.at[pl.ds(0, _NST), pl.ds(col, _C)], slab_v)
    _remap_slab(slab_v)

    iota = lax.iota(jnp.int32, _L)

    def fire_g(ci, buf):
        return [
            pltpu.async_copy(
                proj_hbm.at[slab_v.at[ci, pl.ds(j * _G, _G)]],
                rows_v.at[buf, pl.ds(j * _G, _G)],
                gsem,
            )
            for j in range(_KG)
        ]

    def fire_s(ci, buf):
        for j in range(_KG):
            for k in range(_G // _L):
                bl = k * _L + iota
                s = ci * _KG + j
                si = s >> 2
                dst_v[buf, j, pl.ds(k * _L, _L)] = (
                    base + (si << 9) + (bl << 2) + (s - (si << 2))
                )
        return [
            pltpu.async_copy(
                rows_v.at[buf, pl.ds(j * _G, _G)],
                out_hbm.at[dst_v.at[buf, j]],
                ssem,
            )
            for j in range(_KG)
        ]

    def drain(copies):
        for cp in copies:
            cp.wait()

    drain(fire_g(0, 0))

    def pair(st2, carry):
        c0 = st2 * 2
        ss0 = fire_s(c0, 0)
        gs1 = fire_g(c0 + 1, 1)
        drain(gs1)
        drain(ss0)
        ss1 = fire_s(c0 + 1, 1)
        gs2 = fire_g(c0 + 2, 0)
        drain(gs2)
        drain(ss1)
        return carry

    lax.fori_loop(0, _NST // 2, pair, 0)
    drain(fire_s(_NST - 1, 0))


@functools.cache
def _gather_kernel():
    return pl.kernel(
        _gather_body,
        mesh=plsc.VectorSubcoreMesh(core_axis_name="c", subcore_axis_name="s"),
        compiler_params=pltpu.CompilerParams(use_tc_tiling_on_sc=False),
        out_type=jax.ShapeDtypeStruct((_B, RED), jnp.float32),
        scratch_types=[
            pltpu.VMEM((_NST, _C), jnp.int32),
            pltpu.VMEM((2, _KG, _G), jnp.int32),
            pltpu.VMEM((2, _C, RED), jnp.float32),
            pltpu.SemaphoreType.DMA,
            pltpu.SemaphoreType.DMA,
        ],
    )


_BT = BATCH // VEC
_SR = SEQ * RED
_PB = _B // PACK


def _tr_body(t_ref, o_ref):
    halves = []
    for e in range(2):
        x3 = t_ref[e].reshape(_SR // VEC, VEC, VEC)
        halves.append(x3.transpose(0, 2, 1).reshape(_SR, VEC))
    o_ref[...] = jnp.concatenate(halves, axis=1)


def _transpose(out_flat):
    x = out_flat.reshape(_NBT, _SR, VEC)
    return pl.pallas_call(
        _tr_body,
        grid=(_NBT // 2,),
        in_specs=[pl.BlockSpec((2, _SR, VEC), lambda i: (i, 0, 0))],
        out_specs=pl.BlockSpec((_SR, 2 * VEC), lambda i: (0, i)),
        out_shape=jax.ShapeDtypeStruct((_SR, BATCH), jnp.float32),
    )(x)


def kernel(x, table, W, b):
    proj = _project(table, W, b).reshape(NUM_EMB, RED)
    xq = (
        x.astype(jnp.int32)
        .transpose(1, 0)
        .reshape(_NST, _KG, BATCH // VEC, VEC)
        .transpose(0, 2, 1, 3)
        .reshape(_NST, BATCH // VEC * _KG * VEC)
    )
    out_flat = _gather_kernel()(xq, proj)
    out2 = _transpose(out_flat.reshape(_PB, VEC))
    return out2.reshape(SEQ, RED, BATCH).transpose(2, 0, 1)
